# baseline (speedup 1.0000x reference)
import jax
import jax.numpy as jnp
from jax.experimental import pallas as pl
from jax.experimental.pallas import tpu as pltpu

NUM_CLASSES = 1000


def _onehot_body(xt_ref, out_ref):
    xt = xt_ref[...]
    classes = jax.lax.broadcasted_iota(
        jnp.int32, (2, NUM_CLASSES, xt.shape[2]), 1
    )
    out_ref[...] = (xt.reshape(2, 1, xt.shape[2]) == classes).astype(jnp.float32)


def kernel(x):
    B, S = x.shape
    xt = x.astype(jnp.int32).T
    xt = xt.reshape(S // 2, 2, B)
    out = pl.pallas_call(
        _onehot_body,
        grid=(S // 2,),
        in_specs=[pl.BlockSpec((1, 2, B), lambda s: (s, 0, 0))],
        out_specs=pl.BlockSpec((2, NUM_CLASSES, B), lambda s: (s, 0, 0)),
        out_shape=jax.ShapeDtypeStruct((S, NUM_CLASSES, B), jnp.float32),
        compiler_params=pltpu.CompilerParams(vmem_limit_bytes=128 * 1024 * 1024),
    )(xt)
    return out.transpose(2, 0, 1)


# speedup vs baseline: 1.0034x; 1.0034x over previous
import jax
import jax.numpy as jnp
from jax.experimental import pallas as pl

NUM_CLASSES = 1000


def _onehot_body(xt_ref, out_ref):
    s = pl.program_id(0)
    xt = xt_ref[s]
    classes = jax.lax.broadcasted_iota(
        jnp.int32, (1, NUM_CLASSES, xt.shape[1]), 1
    )
    out_ref[...] = (xt[None] == classes).astype(jnp.float32)


def kernel(x):
    B, S = x.shape
    xt = x.astype(jnp.int32).T
    xt = xt.reshape(S, 1, B)
    out = pl.pallas_call(
        _onehot_body,
        grid=(S,),
        in_specs=[pl.BlockSpec((S, 1, B), lambda s: (0, 0, 0))],
        out_specs=pl.BlockSpec((1, NUM_CLASSES, B), lambda s: (s, 0, 0)),
        out_shape=jax.ShapeDtypeStruct((S, NUM_CLASSES, B), jnp.float32),
    )(xt)
    return out.transpose(2, 0, 1)


# speedup vs baseline: 1.0113x; 1.0078x over previous
import jax
import jax.numpy as jnp
from jax.experimental import pallas as pl

NUM_CLASSES = 1000


def _onehot_body(xt_ref, out_ref):
    xt = xt_ref[...]
    classes = jax.lax.broadcasted_iota(
        jnp.int32, (1, NUM_CLASSES, xt.shape[2]), 1
    )
    out_ref[...] = (xt == classes).astype(jnp.float32)


def kernel(x):
    B, S = x.shape
    xt = x.astype(jnp.int32).T
    xt = xt.reshape(S, 1, B)
    out = pl.pallas_call(
        _onehot_body,
        grid=(S,),
        in_specs=[pl.BlockSpec((1, 1, B), lambda s: (s, 0, 0))],
        out_specs=pl.BlockSpec((1, NUM_CLASSES, B), lambda s: (s, 0, 0)),
        out_shape=jax.ShapeDtypeStruct((S, NUM_CLASSES, B), jnp.float32),
    )(xt)
    return out.transpose(2, 0, 1)
